# baseline (speedup 1.0000x reference)
import functools

import jax
import jax.numpy as jnp
from jax import lax
from jax.experimental import pallas as pl
from jax.experimental.pallas import tpu as pltpu
from jax.experimental.pallas import tpu_sc as plsc

N = 320000
D = 128
S = 10000
NC = 2
NS = 16
NW = NC * NS
RT = N // NW
C = 128
NCHUNK = RT // C
TAIL = RT - NCHUNK * C
WB = 624
WLAST = S - WB * (NS - 1)

_LANES = 16


def _fill(ref, rows, cols, value):
    vec = jnp.full((_LANES,), value, jnp.float32)

    @pl.loop(0, rows)
    def _(r):
        for l in range(cols // _LANES):
            ref[r, pl.ds(l * _LANES, _LANES)] = vec


_sc_mesh = plsc.VectorSubcoreMesh(
    core_axis_name="c", subcore_axis_name="s", num_cores=NC, num_subcores=NS
)


@functools.partial(
    pl.kernel,
    out_type=jax.ShapeDtypeStruct((NC, S, D), jnp.float32),
    mesh=_sc_mesh,
    scratch_types=[
        pltpu.VMEM_SHARED((S, D), jnp.float32),
        pltpu.VMEM((C, D), jnp.float32),
        pltpu.VMEM((C, D), jnp.float32),
        pltpu.VMEM((C, D), jnp.float32),
        pltpu.VMEM((C,), jnp.int32),
        pltpu.VMEM((C,), jnp.int32),
        pltpu.VMEM((C,), jnp.int32),
        pltpu.VMEM((TAIL,), jnp.int32),
        pltpu.SemaphoreType.DMA,
        pltpu.SemaphoreType.DMA,
        pltpu.SemaphoreType.DMA,
        pltpu.SemaphoreType.DMA,
        pltpu.SemaphoreType.DMA,
        pltpu.SemaphoreType.DMA,
        pltpu.SemaphoreType.DMA,
        pltpu.SemaphoreType.DMA,
        pltpu.SemaphoreType.DMA,
        pltpu.SemaphoreType.DMA,
    ],
    compiler_params=pltpu.CompilerParams(use_tc_tiling_on_sc=False),
)
def _sc_aggregate(x_hbm, ids_hbm, psum_hbm,
                  acc_sum, xb0, xb1, xb2, ib0, ib1, ib2, it,
                  gx0, gx1, gx2, gi0, gi1, gi2, sx0, sx1, sx2, wsem):
    cid = lax.axis_index("c")
    sid = lax.axis_index("s")
    row0 = (cid * NS + sid) * RT
    base = sid * WB

    xb = (xb0, xb1, xb2)
    ib = (ib0, ib1, ib2)
    gx = (gx0, gx1, gx2)
    gi = (gi0, gi1, gi2)
    sx = (sx0, sx1, sx2)

    def fire_gather(chunk, j):
        r = row0 + chunk * C
        pltpu.async_copy(x_hbm.at[pl.ds(r, C)], xb[j], gx[j])
        pltpu.async_copy(ids_hbm.at[pl.ds(r, C)], ib[j], gi[j])

    def wait_gather(chunk, j):
        r = row0 + chunk * C
        pltpu.make_async_copy(x_hbm.at[pl.ds(r, C)], xb[j], gx[j]).wait()
        pltpu.make_async_copy(ids_hbm.at[pl.ds(r, C)], ib[j], gi[j]).wait()

    def fire_scatter(j):
        pltpu.async_copy(xb[j], acc_sum.at[ib[j]], sx[j], add=True)

    def wait_scatter(j):
        pltpu.make_async_copy(xb[j], acc_sum.at[ib[j]], sx[j]).wait()

    _fill(xb0, C, D, 0.0)

    def zero_region(nfull, rem):
        cps = []
        for k in range(nfull):
            cps.append(pltpu.async_copy(
                xb0, acc_sum.at[pl.ds(base + k * C, C)], wsem))
        if rem:
            cps.append(pltpu.async_copy(
                xb0.at[pl.ds(0, rem)],
                acc_sum.at[pl.ds(base + nfull * C, rem)], wsem))
        for cp in cps:
            cp.wait()

    @pl.when(sid < NS - 1)
    def _():
        zero_region(WB // C, WB % C)

    @pl.when(sid == NS - 1)
    def _():
        zero_region(WLAST // C, WLAST % C)

    fire_gather(0, 0)
    fire_gather(1, 1)

    plsc.subcore_barrier()


    wait_gather(0, 0)
    fire_scatter(0)
    fire_gather(2, 2)
    wait_gather(1, 1)
    fire_scatter(1)
    wait_scatter(0)
    fire_gather(3, 0)
    wait_gather(2, 2)
    fire_scatter(2)
    wait_scatter(1)
    fire_gather(4, 1)

    @pl.loop(0, (NCHUNK - 6) // 3)
    def _(k):
        c0 = 3 + 3 * k
        for j in range(3):
            cc = c0 + j
            wait_gather(cc, j)
            fire_scatter(j)
            wait_scatter((j + 2) % 3)
            fire_gather(cc + 2, (j + 2) % 3)

    wait_gather(NCHUNK - 3, 0)
    fire_scatter(0)
    wait_scatter(2)
    fire_gather(NCHUNK - 1, 2)
    wait_gather(NCHUNK - 2, 1)
    fire_scatter(1)
    wait_scatter(0)
    wait_gather(NCHUNK - 1, 2)
    fire_scatter(2)
    wait_scatter(1)
    wait_scatter(2)

    rt = row0 + NCHUNK * C
    pltpu.sync_copy(x_hbm.at[pl.ds(rt, TAIL)], xb0.at[pl.ds(0, TAIL)])
    pltpu.sync_copy(ids_hbm.at[pl.ds(rt, TAIL)], it)
    pltpu.sync_copy(xb0.at[pl.ds(0, TAIL)], acc_sum.at[it], add=True)

    plsc.subcore_barrier()

    def write_out(nrows):
        pltpu.async_copy(
            acc_sum.at[pl.ds(base, nrows)],
            psum_hbm.at[cid, pl.ds(base, nrows)], wsem).wait()

    @pl.when(sid < NS - 1)
    def _():
        write_out(WB)

    @pl.when(sid == NS - 1)
    def _():
        write_out(WLAST)


_R = 640
_NBLK = N // _R
_W = 128
_NWIN = -(-S // _W)


def _counts_body(ids_smem, ids_ref, o_ref, acc):
    pid = pl.program_id(0)

    @pl.when(pid == 0)
    def _():
        acc[...] = jnp.zeros_like(acc)

    ids_col = ids_ref[0]
    lo = ids_smem[0, 0, 0]
    hi = ids_smem[0, 0, _R - 1]
    lane = lax.broadcasted_iota(jnp.int32, (_R, _W), 1)

    def body(w, carry):
        onehot = (lane + w * _W == ids_col).astype(jnp.float32)
        cnt = jnp.sum(onehot, axis=0, keepdims=True)
        acc[pl.ds(w, 1), :] += cnt
        return carry

    lax.fori_loop(lo // _W, hi // _W + 1, body, 0)

    @pl.when(pid == _NBLK - 1)
    def _():
        o_ref[...] = acc[...]


_counts_tc = pl.pallas_call(
    _counts_body,
    grid=(_NBLK,),
    in_specs=[
        pl.BlockSpec((1, 1, _R), lambda i: (i, 0, 0),
                     memory_space=pltpu.SMEM),
        pl.BlockSpec((1, _R, 1), lambda i: (i, 0, 0)),
    ],
    out_specs=pl.BlockSpec((_NWIN + 1, _W), lambda i: (0, 0)),
    out_shape=jax.ShapeDtypeStruct((_NWIN + 1, _W), jnp.float32),
    scratch_shapes=[pltpu.VMEM((_NWIN + 1, _W), jnp.float32)],
)


_BLK = 1000


def _combine_body(ps_ref, cnt_ref, o_ref):
    sums = ps_ref[0] + ps_ref[1]
    o_ref[...] = sums / jnp.maximum(cnt_ref[...], 1.0)


_combine = pl.pallas_call(
    _combine_body,
    grid=(S // _BLK,),
    in_specs=[
        pl.BlockSpec((NC, _BLK, D), lambda i: (0, i, 0)),
        pl.BlockSpec((_BLK, 1), lambda i: (i, 0)),
    ],
    out_specs=pl.BlockSpec((_BLK, D), lambda i: (i, 0)),
    out_shape=jax.ShapeDtypeStruct((S, D), jnp.float32),
)


def kernel(x, segment_ids):
    ids_smem = segment_ids.reshape(_NBLK, 1, _R)
    ids_col = segment_ids.reshape(_NBLK, _R, 1)
    psum = _sc_aggregate(x, segment_ids)
    cnt_win = _counts_tc(ids_smem, ids_col)
    cnt = cnt_win.reshape(-1)[:S].reshape(S, 1)
    return _combine(psum, cnt)


# speedup vs baseline: 3.9187x; 3.9187x over previous
import functools

import jax
import jax.numpy as jnp
from jax import lax
from jax.experimental import pallas as pl
from jax.experimental.pallas import tpu as pltpu
from jax.experimental.pallas import tpu_sc as plsc

N = 320000
D = 128
S = 10000
NC = 2
NS = 16
NW = NC * NS
RT = N // NW
C = 96
NCHUNK = RT // C
TAIL = RT - NCHUNK * C
WB = 624
WLAST = S - WB * (NS - 1)
CW = 16

_LANES = 16


def _fill(ref, rows, cols, value):
    vec = jnp.full((_LANES,), value, jnp.float32)

    @pl.loop(0, rows)
    def _(r):
        for l in range(cols // _LANES):
            ref[r, pl.ds(l * _LANES, _LANES)] = vec


_sc_mesh = plsc.VectorSubcoreMesh(
    core_axis_name="c", subcore_axis_name="s", num_cores=NC, num_subcores=NS
)


@functools.partial(
    pl.kernel,
    out_type=(
        jax.ShapeDtypeStruct((NC, S, D), jnp.float32),
        jax.ShapeDtypeStruct((NC, S, CW), jnp.float32),
    ),
    mesh=_sc_mesh,
    scratch_types=[
        pltpu.VMEM_SHARED((S, D), jnp.float32),
        pltpu.VMEM_SHARED((S, CW), jnp.float32),
        pltpu.VMEM((C, D), jnp.float32),
        pltpu.VMEM((C, D), jnp.float32),
        pltpu.VMEM((C, D), jnp.float32),
        pltpu.VMEM((C,), jnp.int32),
        pltpu.VMEM((C,), jnp.int32),
        pltpu.VMEM((C,), jnp.int32),
        pltpu.VMEM((C, CW), jnp.float32),
        pltpu.VMEM((C, CW), jnp.float32),
        pltpu.VMEM((TAIL,), jnp.int32),
        pltpu.SemaphoreType.DMA,
        pltpu.SemaphoreType.DMA,
        pltpu.SemaphoreType.DMA,
        pltpu.SemaphoreType.DMA,
        pltpu.SemaphoreType.DMA,
        pltpu.SemaphoreType.DMA,
        pltpu.SemaphoreType.DMA,
        pltpu.SemaphoreType.DMA,
        pltpu.SemaphoreType.DMA,
        pltpu.SemaphoreType.DMA,
        pltpu.SemaphoreType.DMA,
        pltpu.SemaphoreType.DMA,
        pltpu.SemaphoreType.DMA,
    ],
    compiler_params=pltpu.CompilerParams(use_tc_tiling_on_sc=False),
)
def _sc_aggregate(x_hbm, ids_hbm, psum_hbm, pcnt_hbm,
                  acc_sum, acc_cnt, xb0, xb1, xb2, ib0, ib1, ib2,
                  ones, zc, it,
                  gx0, gx1, gx2, gi0, gi1, gi2,
                  sx0, sx1, sx2, sc0, sc1, sc2, wsem):
    cid = lax.axis_index("c")
    sid = lax.axis_index("s")
    row0 = (cid * NS + sid) * RT
    base = sid * WB

    xb = (xb0, xb1, xb2)
    ib = (ib0, ib1, ib2)
    gx = (gx0, gx1, gx2)
    gi = (gi0, gi1, gi2)
    sx = (sx0, sx1, sx2)
    sc = (sc0, sc1, sc2)

    def fire_gather(chunk, j):
        r = row0 + chunk * C
        pltpu.async_copy(x_hbm.at[pl.ds(r, C)], xb[j], gx[j])
        pltpu.async_copy(ids_hbm.at[pl.ds(r, C)], ib[j], gi[j])

    def wait_gather(chunk, j):
        r = row0 + chunk * C
        pltpu.make_async_copy(x_hbm.at[pl.ds(r, C)], xb[j], gx[j]).wait()
        pltpu.make_async_copy(ids_hbm.at[pl.ds(r, C)], ib[j], gi[j]).wait()

    def fire_scatter(j):
        pltpu.async_copy(xb[j], acc_sum.at[ib[j]], sx[j], add=True)
        pltpu.async_copy(ones, acc_cnt.at[ib[j]], sc[j], add=True)

    def wait_scatter(j):
        pltpu.make_async_copy(xb[j], acc_sum.at[ib[j]], sx[j]).wait()
        pltpu.make_async_copy(ones, acc_cnt.at[ib[j]], sc[j]).wait()

    _fill(xb0, C, D, 0.0)
    _fill(zc, C, CW, 0.0)
    _fill(ones, C, CW, 1.0)

    def zero_region(nfull, rem):
        cps = []
        for k in range(nfull):
            cps.append(pltpu.async_copy(
                xb0, acc_sum.at[pl.ds(base + k * C, C)], wsem))
            cps.append(pltpu.async_copy(
                zc, acc_cnt.at[pl.ds(base + k * C, C)], wsem))
        if rem:
            cps.append(pltpu.async_copy(
                xb0.at[pl.ds(0, rem)],
                acc_sum.at[pl.ds(base + nfull * C, rem)], wsem))
            cps.append(pltpu.async_copy(
                zc.at[pl.ds(0, rem)],
                acc_cnt.at[pl.ds(base + nfull * C, rem)], wsem))
        for cp in cps:
            cp.wait()

    @pl.when(sid < NS - 1)
    def _():
        zero_region(WB // C, WB % C)

    @pl.when(sid == NS - 1)
    def _():
        zero_region(WLAST // C, WLAST % C)

    fire_gather(0, 0)
    fire_gather(1, 1)

    plsc.subcore_barrier()


    wait_gather(0, 0)
    fire_scatter(0)
    fire_gather(2, 2)
    wait_gather(1, 1)
    fire_scatter(1)
    wait_scatter(0)
    fire_gather(3, 0)
    wait_gather(2, 2)
    fire_scatter(2)
    wait_scatter(1)
    fire_gather(4, 1)

    @pl.loop(0, (NCHUNK - 5) // 3)
    def _(k):
        c0 = 3 + 3 * k
        for j in range(3):
            cc = c0 + j
            wait_gather(cc, j)
            fire_scatter(j)
            wait_scatter((j + 2) % 3)
            fire_gather(cc + 2, (j + 2) % 3)

    j2 = (NCHUNK - 2) % 3
    j1 = (NCHUNK - 1) % 3
    wait_gather(NCHUNK - 2, j2)
    fire_scatter(j2)
    wait_scatter((j2 + 2) % 3)
    wait_gather(NCHUNK - 1, j1)
    fire_scatter(j1)
    wait_scatter(j2)
    wait_scatter(j1)

    rt = row0 + NCHUNK * C
    pltpu.sync_copy(x_hbm.at[pl.ds(rt, TAIL)], xb0.at[pl.ds(0, TAIL)])
    pltpu.sync_copy(ids_hbm.at[pl.ds(rt, TAIL)], it)
    pltpu.sync_copy(xb0.at[pl.ds(0, TAIL)], acc_sum.at[it], add=True)
    pltpu.sync_copy(ones.at[pl.ds(0, TAIL)], acc_cnt.at[it], add=True)

    plsc.subcore_barrier()

    def write_out(nrows):
        w1 = pltpu.async_copy(
            acc_sum.at[pl.ds(base, nrows)],
            psum_hbm.at[cid, pl.ds(base, nrows)], wsem)
        w2 = pltpu.async_copy(
            acc_cnt.at[pl.ds(base, nrows)],
            pcnt_hbm.at[cid, pl.ds(base, nrows)], wsem)
        w1.wait()
        w2.wait()

    @pl.when(sid < NS - 1)
    def _():
        write_out(WB)

    @pl.when(sid == NS - 1)
    def _():
        write_out(WLAST)


_BLK = 1000


def _combine_body(ps_ref, pc_ref, o_ref):
    sums = ps_ref[0] + ps_ref[1]
    cnts = pc_ref[0] + pc_ref[1]
    cnt = jnp.maximum(cnts[:, 0:1], 1.0)
    o_ref[...] = sums / cnt


_combine = pl.pallas_call(
    _combine_body,
    grid=(S // _BLK,),
    in_specs=[
        pl.BlockSpec((NC, _BLK, D), lambda i: (0, i, 0)),
        pl.BlockSpec((NC, _BLK, CW), lambda i: (0, i, 0)),
    ],
    out_specs=pl.BlockSpec((_BLK, D), lambda i: (i, 0)),
    out_shape=jax.ShapeDtypeStruct((S, D), jnp.float32),
)


def kernel(x, segment_ids):
    psum, pcnt = _sc_aggregate(x, segment_ids)
    return _combine(psum, pcnt)


# speedup vs baseline: 3.9320x; 1.0034x over previous
import functools

import jax
import jax.numpy as jnp
from jax import lax
from jax.experimental import pallas as pl
from jax.experimental.pallas import tpu as pltpu
from jax.experimental.pallas import tpu_sc as plsc

N = 320000
D = 128
S = 10000
NC = 2
NS = 16
NW = NC * NS
RT = N // NW
C = 96
NCHUNK = RT // C
TAIL = RT - NCHUNK * C
WB = 624
WLAST = S - WB * (NS - 1)
CW = 16

_LANES = 16


def _fill(ref, rows, cols, value):
    vec = jnp.full((_LANES,), value, jnp.float32)

    @pl.loop(0, rows)
    def _(r):
        for l in range(cols // _LANES):
            ref[r, pl.ds(l * _LANES, _LANES)] = vec


_sc_mesh = plsc.VectorSubcoreMesh(
    core_axis_name="c", subcore_axis_name="s", num_cores=NC, num_subcores=NS
)


@functools.partial(
    pl.kernel,
    out_type=(
        jax.ShapeDtypeStruct((NC, S, D), jnp.float32),
        jax.ShapeDtypeStruct((NC, S, CW), jnp.float32),
    ),
    mesh=_sc_mesh,
    scratch_types=[
        pltpu.VMEM_SHARED((S, D), jnp.float32),
        pltpu.VMEM_SHARED((S, CW), jnp.float32),
        pltpu.VMEM((C, D), jnp.float32),
        pltpu.VMEM((C, D), jnp.float32),
        pltpu.VMEM((C, D), jnp.float32),
        pltpu.VMEM((C,), jnp.int32),
        pltpu.VMEM((C,), jnp.int32),
        pltpu.VMEM((C,), jnp.int32),
        pltpu.VMEM((C, CW), jnp.float32),
        pltpu.VMEM((C, CW), jnp.float32),
        pltpu.VMEM((TAIL,), jnp.int32),
        pltpu.SemaphoreType.DMA,
        pltpu.SemaphoreType.DMA,
        pltpu.SemaphoreType.DMA,
        pltpu.SemaphoreType.DMA,
        pltpu.SemaphoreType.DMA,
        pltpu.SemaphoreType.DMA,
        pltpu.SemaphoreType.DMA,
        pltpu.SemaphoreType.DMA,
        pltpu.SemaphoreType.DMA,
        pltpu.SemaphoreType.DMA,
        pltpu.SemaphoreType.DMA,
        pltpu.SemaphoreType.DMA,
        pltpu.SemaphoreType.DMA,
    ],
    compiler_params=pltpu.CompilerParams(use_tc_tiling_on_sc=False),
)
def _sc_aggregate(x_hbm, ids_hbm, psum_hbm, pcnt_hbm,
                  acc_sum, acc_cnt, xb0, xb1, xb2, ib0, ib1, ib2,
                  ones, zc, it,
                  gx0, gx1, gx2, gi0, gi1, gi2,
                  sx0, sx1, sx2, sc0, sc1, sc2, wsem):
    cid = lax.axis_index("c")
    sid = lax.axis_index("s")
    row0 = (cid * NS + sid) * RT
    base = sid * WB

    xb = (xb0, xb1, xb2)
    ib = (ib0, ib1, ib2)
    gx = (gx0, gx1, gx2)
    gi = (gi0, gi1, gi2)
    sx = (sx0, sx1, sx2)
    sc = (sc0, sc1, sc2)

    def fire_gather(chunk, j):
        r = row0 + chunk * C
        pltpu.async_copy(x_hbm.at[pl.ds(r, C)], xb[j], gx[j])
        pltpu.async_copy(ids_hbm.at[pl.ds(r, C)], ib[j], gi[j])

    def wait_gather(chunk, j):
        r = row0 + chunk * C
        pltpu.make_async_copy(x_hbm.at[pl.ds(r, C)], xb[j], gx[j]).wait()
        pltpu.make_async_copy(ids_hbm.at[pl.ds(r, C)], ib[j], gi[j]).wait()

    def fire_scatter(j):
        pltpu.async_copy(xb[j], acc_sum.at[ib[j]], sx[j], add=True)
        pltpu.async_copy(ones, acc_cnt.at[ib[j]], sc[j], add=True)

    def wait_scatter(j):
        pltpu.make_async_copy(xb[j], acc_sum.at[ib[j]], sx[j]).wait()
        pltpu.make_async_copy(ones, acc_cnt.at[ib[j]], sc[j]).wait()

    fire_gather(1, 1)
    fire_gather(2, 2)

    _fill(xb0, C, D, 0.0)
    _fill(zc, C, CW, 0.0)
    _fill(ones, C, CW, 1.0)

    def zero_region(nfull, rem):
        cps = []
        for k in range(nfull):
            cps.append(pltpu.async_copy(
                xb0, acc_sum.at[pl.ds(base + k * C, C)], wsem))
            cps.append(pltpu.async_copy(
                zc, acc_cnt.at[pl.ds(base + k * C, C)], wsem))
        if rem:
            cps.append(pltpu.async_copy(
                xb0.at[pl.ds(0, rem)],
                acc_sum.at[pl.ds(base + nfull * C, rem)], wsem))
            cps.append(pltpu.async_copy(
                zc.at[pl.ds(0, rem)],
                acc_cnt.at[pl.ds(base + nfull * C, rem)], wsem))
        for cp in cps:
            cp.wait()

    @pl.when(sid < NS - 1)
    def _():
        zero_region(WB // C, WB % C)

    @pl.when(sid == NS - 1)
    def _():
        zero_region(WLAST // C, WLAST % C)

    fire_gather(0, 0)

    plsc.subcore_barrier()


    wait_gather(0, 0)
    fire_scatter(0)
    wait_gather(1, 1)
    fire_scatter(1)
    wait_scatter(0)
    fire_gather(3, 0)
    wait_gather(2, 2)
    fire_scatter(2)
    wait_scatter(1)
    fire_gather(4, 1)

    @pl.loop(0, (NCHUNK - 5) // 3)
    def _(k):
        c0 = 3 + 3 * k
        for j in range(3):
            cc = c0 + j
            wait_gather(cc, j)
            fire_scatter(j)
            wait_scatter((j + 2) % 3)
            fire_gather(cc + 2, (j + 2) % 3)

    j2 = (NCHUNK - 2) % 3
    j1 = (NCHUNK - 1) % 3
    wait_gather(NCHUNK - 2, j2)
    fire_scatter(j2)
    wait_scatter((j2 + 2) % 3)
    wait_gather(NCHUNK - 1, j1)
    fire_scatter(j1)
    wait_scatter(j2)
    wait_scatter(j1)

    rt = row0 + NCHUNK * C
    pltpu.sync_copy(x_hbm.at[pl.ds(rt, TAIL)], xb0.at[pl.ds(0, TAIL)])
    pltpu.sync_copy(ids_hbm.at[pl.ds(rt, TAIL)], it)
    pltpu.sync_copy(xb0.at[pl.ds(0, TAIL)], acc_sum.at[it], add=True)
    pltpu.sync_copy(ones.at[pl.ds(0, TAIL)], acc_cnt.at[it], add=True)

    plsc.subcore_barrier()

    def write_out(nrows):
        w1 = pltpu.async_copy(
            acc_sum.at[pl.ds(base, nrows)],
            psum_hbm.at[cid, pl.ds(base, nrows)], wsem)
        w2 = pltpu.async_copy(
            acc_cnt.at[pl.ds(base, nrows)],
            pcnt_hbm.at[cid, pl.ds(base, nrows)], wsem)
        w1.wait()
        w2.wait()

    @pl.when(sid < NS - 1)
    def _():
        write_out(WB)

    @pl.when(sid == NS - 1)
    def _():
        write_out(WLAST)


_BLK = 1000


def _combine_body(ps_ref, pc_ref, o_ref):
    sums = ps_ref[0] + ps_ref[1]
    cnts = pc_ref[0] + pc_ref[1]
    cnt = jnp.maximum(cnts[:, 0:1], 1.0)
    o_ref[...] = sums / cnt


_combine = pl.pallas_call(
    _combine_body,
    grid=(S // _BLK,),
    in_specs=[
        pl.BlockSpec((NC, _BLK, D), lambda i: (0, i, 0)),
        pl.BlockSpec((NC, _BLK, CW), lambda i: (0, i, 0)),
    ],
    out_specs=pl.BlockSpec((_BLK, D), lambda i: (i, 0)),
    out_shape=jax.ShapeDtypeStruct((S, D), jnp.float32),
)


def kernel(x, segment_ids):
    psum, pcnt = _sc_aggregate(x, segment_ids)
    return _combine(psum, pcnt)


# speedup vs baseline: 3.9376x; 1.0014x over previous
import functools

import jax
import jax.numpy as jnp
from jax import lax
from jax.experimental import pallas as pl
from jax.experimental.pallas import tpu as pltpu
from jax.experimental.pallas import tpu_sc as plsc

N = 320000
D = 128
S = 10000
NC = 2
NS = 16
NW = NC * NS
RT = N // NW
C = 96
NCHUNK = RT // C
TAIL = RT - NCHUNK * C
WB = 624
WLAST = S - WB * (NS - 1)
CW = 16

_LANES = 16


_sc_mesh = plsc.VectorSubcoreMesh(
    core_axis_name="c", subcore_axis_name="s", num_cores=NC, num_subcores=NS
)


@functools.partial(
    pl.kernel,
    out_type=(
        jax.ShapeDtypeStruct((NC, S, D), jnp.float32),
        jax.ShapeDtypeStruct((NC, S, CW), jnp.float32),
    ),
    mesh=_sc_mesh,
    scratch_types=[
        pltpu.VMEM_SHARED((S, D), jnp.float32),
        pltpu.VMEM_SHARED((S, CW), jnp.float32),
        pltpu.VMEM((C, D), jnp.float32),
        pltpu.VMEM((C, D), jnp.float32),
        pltpu.VMEM((C, D), jnp.float32),
        pltpu.VMEM((C,), jnp.int32),
        pltpu.VMEM((C,), jnp.int32),
        pltpu.VMEM((C,), jnp.int32),
        pltpu.VMEM((C, CW), jnp.float32),
        pltpu.VMEM((C, CW), jnp.float32),
        pltpu.VMEM((TAIL,), jnp.int32),
        pltpu.SemaphoreType.DMA,
        pltpu.SemaphoreType.DMA,
        pltpu.SemaphoreType.DMA,
        pltpu.SemaphoreType.DMA,
        pltpu.SemaphoreType.DMA,
        pltpu.SemaphoreType.DMA,
        pltpu.SemaphoreType.DMA,
        pltpu.SemaphoreType.DMA,
        pltpu.SemaphoreType.DMA,
        pltpu.SemaphoreType.DMA,
        pltpu.SemaphoreType.DMA,
        pltpu.SemaphoreType.DMA,
        pltpu.SemaphoreType.DMA,
    ],
    compiler_params=pltpu.CompilerParams(use_tc_tiling_on_sc=False),
)
def _sc_aggregate(x_hbm, ids_hbm, psum_hbm, pcnt_hbm,
                  acc_sum, acc_cnt, xb0, xb1, xb2, ib0, ib1, ib2,
                  ones, zc, it,
                  gx0, gx1, gx2, gi0, gi1, gi2,
                  sx0, sx1, sx2, sc0, sc1, sc2, wsem):
    cid = lax.axis_index("c")
    sid = lax.axis_index("s")
    row0 = (cid * NS + sid) * RT
    base = sid * WB

    xb = (xb0, xb1, xb2)
    ib = (ib0, ib1, ib2)
    gx = (gx0, gx1, gx2)
    gi = (gi0, gi1, gi2)
    sx = (sx0, sx1, sx2)
    sc = (sc0, sc1, sc2)

    def fire_gather(chunk, j):
        r = row0 + chunk * C
        pltpu.async_copy(x_hbm.at[pl.ds(r, C)], xb[j], gx[j])
        pltpu.async_copy(ids_hbm.at[pl.ds(r, C)], ib[j], gi[j])

    def wait_gather(chunk, j):
        r = row0 + chunk * C
        pltpu.make_async_copy(x_hbm.at[pl.ds(r, C)], xb[j], gx[j]).wait()
        pltpu.make_async_copy(ids_hbm.at[pl.ds(r, C)], ib[j], gi[j]).wait()

    def fire_scatter(j):
        pltpu.async_copy(xb[j], acc_sum.at[ib[j]], sx[j], add=True)
        pltpu.async_copy(ones, acc_cnt.at[ib[j]], sc[j], add=True)

    def wait_scatter(j):
        pltpu.make_async_copy(xb[j], acc_sum.at[ib[j]], sx[j]).wait()
        pltpu.make_async_copy(ones, acc_cnt.at[ib[j]], sc[j]).wait()

    fire_gather(1, 1)
    fire_gather(2, 2)

    zeros16 = jnp.zeros((_LANES,), jnp.float32)
    ones16 = jnp.ones((_LANES,), jnp.float32)

    @pl.loop(0, C)
    def _(r):
        for l in range(D // _LANES):
            xb0[r, pl.ds(l * _LANES, _LANES)] = zeros16
        zc[r, pl.ds(0, CW)] = zeros16
        ones[r, pl.ds(0, CW)] = ones16

    def zero_region(nfull, rem):
        cps = []
        for k in range(nfull):
            cps.append(pltpu.async_copy(
                xb0, acc_sum.at[pl.ds(base + k * C, C)], wsem))
            cps.append(pltpu.async_copy(
                zc, acc_cnt.at[pl.ds(base + k * C, C)], wsem))
        if rem:
            cps.append(pltpu.async_copy(
                xb0.at[pl.ds(0, rem)],
                acc_sum.at[pl.ds(base + nfull * C, rem)], wsem))
            cps.append(pltpu.async_copy(
                zc.at[pl.ds(0, rem)],
                acc_cnt.at[pl.ds(base + nfull * C, rem)], wsem))
        for cp in cps:
            cp.wait()

    @pl.when(sid < NS - 1)
    def _():
        zero_region(WB // C, WB % C)

    @pl.when(sid == NS - 1)
    def _():
        zero_region(WLAST // C, WLAST % C)

    fire_gather(0, 0)

    plsc.subcore_barrier()


    wait_gather(0, 0)
    fire_scatter(0)
    wait_gather(1, 1)
    fire_scatter(1)
    wait_scatter(0)
    fire_gather(3, 0)
    wait_gather(2, 2)
    fire_scatter(2)
    wait_scatter(1)
    fire_gather(4, 1)

    @pl.loop(0, (NCHUNK - 5) // 3)
    def _(k):
        c0 = 3 + 3 * k
        for j in range(3):
            cc = c0 + j
            wait_gather(cc, j)
            fire_scatter(j)
            wait_scatter((j + 2) % 3)
            fire_gather(cc + 2, (j + 2) % 3)

    j2 = (NCHUNK - 2) % 3
    j1 = (NCHUNK - 1) % 3
    wait_gather(NCHUNK - 2, j2)
    fire_scatter(j2)
    wait_scatter((j2 + 2) % 3)
    wait_gather(NCHUNK - 1, j1)
    fire_scatter(j1)
    wait_scatter(j2)
    wait_scatter(j1)

    rt = row0 + NCHUNK * C
    pltpu.sync_copy(x_hbm.at[pl.ds(rt, TAIL)], xb0.at[pl.ds(0, TAIL)])
    pltpu.sync_copy(ids_hbm.at[pl.ds(rt, TAIL)], it)
    pltpu.sync_copy(xb0.at[pl.ds(0, TAIL)], acc_sum.at[it], add=True)
    pltpu.sync_copy(ones.at[pl.ds(0, TAIL)], acc_cnt.at[it], add=True)

    plsc.subcore_barrier()

    def write_out(nrows):
        w1 = pltpu.async_copy(
            acc_sum.at[pl.ds(base, nrows)],
            psum_hbm.at[cid, pl.ds(base, nrows)], wsem)
        w2 = pltpu.async_copy(
            acc_cnt.at[pl.ds(base, nrows)],
            pcnt_hbm.at[cid, pl.ds(base, nrows)], wsem)
        w1.wait()
        w2.wait()

    @pl.when(sid < NS - 1)
    def _():
        write_out(WB)

    @pl.when(sid == NS - 1)
    def _():
        write_out(WLAST)


_BLK = 1000


def _combine_body(ps_ref, pc_ref, o_ref):
    sums = ps_ref[0] + ps_ref[1]
    cnts = pc_ref[0] + pc_ref[1]
    cnt = jnp.maximum(cnts[:, 0:1], 1.0)
    o_ref[...] = sums / cnt


_combine = pl.pallas_call(
    _combine_body,
    grid=(S // _BLK,),
    in_specs=[
        pl.BlockSpec((NC, _BLK, D), lambda i: (0, i, 0)),
        pl.BlockSpec((NC, _BLK, CW), lambda i: (0, i, 0)),
    ],
    out_specs=pl.BlockSpec((_BLK, D), lambda i: (i, 0)),
    out_shape=jax.ShapeDtypeStruct((S, D), jnp.float32),
)


def kernel(x, segment_ids):
    psum, pcnt = _sc_aggregate(x, segment_ids)
    return _combine(psum, pcnt)
